# baseline (speedup 1.0000x reference)
import functools

import numpy as np
import jax
import jax.numpy as jnp
from jax import lax
from jax.experimental import pallas as pl
from jax.experimental.pallas import tpu as pltpu
from jax.experimental.pallas import tpu_sc as plsc

_B, _T, _D, _SEG, _R = 64, 512, 256, 5, 100
_MASK_RATE = 0.15
_NTOK = _SEG * _T + 7
_OFFS = (2, 515, 1028, 1541, 2054)
_SPECIALS = ((0, 4), (1, 1), (514, 3), (1027, 3), (1540, 3), (2053, 3), (2566, 2))
_CHUNK = 128

_PLAN_CACHE = None

_ROT1 = (13, 15, 26, 6)
_ROT2 = (17, 29, 16, 24)


def _rotl(x, d):
    return ((x << np.uint32(d)) | (x >> np.uint32(32 - d))).astype(np.uint32)


def _tf_pair(key, c1, c2):
    x0 = c1.astype(np.uint32).copy()
    x1 = c2.astype(np.uint32).copy()
    ks0, ks1 = np.uint32(key[0]), np.uint32(key[1])
    ks2 = np.uint32(ks0 ^ ks1 ^ np.uint32(0x1BD11BDA))
    x0 += ks0
    x1 += ks1

    def rounds(x0, x1, rots):
        for r in rots:
            x0 = (x0 + x1).astype(np.uint32)
            x1 = _rotl(x1, r)
            x1 = x1 ^ x0
        return x0, x1

    x0, x1 = rounds(x0, x1, _ROT1)
    x0 = (x0 + ks1).astype(np.uint32); x1 = (x1 + ks2 + np.uint32(1)).astype(np.uint32)
    x0, x1 = rounds(x0, x1, _ROT2)
    x0 = (x0 + ks2).astype(np.uint32); x1 = (x1 + ks0 + np.uint32(2)).astype(np.uint32)
    x0, x1 = rounds(x0, x1, _ROT1)
    x0 = (x0 + ks0).astype(np.uint32); x1 = (x1 + ks1 + np.uint32(3)).astype(np.uint32)
    x0, x1 = rounds(x0, x1, _ROT2)
    x0 = (x0 + ks1).astype(np.uint32); x1 = (x1 + ks2 + np.uint32(4)).astype(np.uint32)
    x0, x1 = rounds(x0, x1, _ROT1)
    x0 = (x0 + ks2).astype(np.uint32); x1 = (x1 + ks0 + np.uint32(5)).astype(np.uint32)
    return x0, x1


def _np_split(key, num):
    i = np.arange(num, dtype=np.uint64)
    b1, b2 = _tf_pair(key, (i >> np.uint64(32)).astype(np.uint32), i.astype(np.uint32))
    return np.stack([b1, b2], axis=1)


def _np_random_bits(key, shape):
    n = int(np.prod(shape))
    i = np.arange(n, dtype=np.uint64)
    b1, b2 = _tf_pair(key, (i >> np.uint64(32)).astype(np.uint32), i.astype(np.uint32))
    return (b1 ^ b2).reshape(shape)


def _np_uniform(key, shape):
    bits = _np_random_bits(key, shape)
    f = ((bits >> np.uint32(9)) | np.uint32(0x3F800000)).view(np.float32) - np.float32(1.0)
    return np.maximum(np.float32(0.0), f)


def _np_randint(key, shape, minval, maxval):
    k1, k2 = _np_split(key, 2)
    higher = _np_random_bits(k1, shape)
    lower = _np_random_bits(k2, shape)
    span = np.uint32(maxval - minval)
    multiplier = np.uint32((2 ** 16) % int(span))
    multiplier = np.uint32((int(multiplier) * int(multiplier)) % int(span))
    offset = ((higher % span) * multiplier + lower % span) % span
    return (minval + offset).astype(np.int32)


def _plan():
    global _PLAN_CACHE
    if _PLAN_CACHE is not None:
        return _PLAN_CACHE
    ks = _np_split(np.array([0, 42], np.uint32), 4)
    u1 = _np_uniform(ks[0], (_SEG, _T))
    u2 = _np_uniform(ks[1], (_SEG, _T))
    u3 = _np_uniform(ks[2], (_SEG, _T))
    ridx = _np_randint(ks[3], (_SEG, _T), 0, _R)
    replace = (u1 < _MASK_RATE) & (np.arange(_T)[None, :] < _T - 1)
    choice = np.where(u2 < 0.8, 0, np.where(u3 < 0.5, 1, 2))
    overwrite = replace & (choice != 1)
    tblrow = np.where(choice == 0, 5, 6 + ridx)

    bidx = np.arange(_B)
    groups = []
    for s in range(_SEG):
        kept_t = np.nonzero(~overwrite[s])[0]
        src = (bidx[:, None] * _T + kept_t[None, :]).ravel()
        dst = (bidx[:, None] * _NTOK + _OFFS[s] + kept_t[None, :]).ravel()
        groups.append((src, dst))

    js = [j for j, _ in _SPECIALS]
    ks = [kk for _, kk in _SPECIALS]
    for s in range(_SEG):
        ot = np.nonzero(overwrite[s])[0]
        js += list(_OFFS[s] + ot)
        ks += list(tblrow[s][ot])
    js = np.array(js)
    ks = np.array(ks)
    groups.append((np.tile(ks, _B), (bidx[:, None] * _NTOK + js[None, :]).ravel()))

    all_chunks, bases, counts = [], [], []
    base = 0
    for src, dst in groups:
        pad = (-len(src)) % _CHUNK
        src = np.concatenate([src, np.full(pad, src[-1])])
        dst = np.concatenate([dst, np.full(pad, dst[-1])])
        nch = len(src) // _CHUNK
        all_chunks.append(
            np.stack([src, dst], 0).reshape(2, nch, _CHUNK).transpose(1, 0, 2))
        bases.append(base)
        counts.append(nch)
        base += nch
    idx = np.ascontiguousarray(np.concatenate(all_chunks, 0).astype(np.int32))

    seg_index = np.array([0] * 515 + sum(([s] * 513 for s in range(1, 5)), []),
                         dtype=np.int32)
    _PLAN_CACHE = (idx, tuple(bases), tuple(counts), seg_index)
    return _PLAN_CACHE


@functools.lru_cache(maxsize=1)
def _make_sc_kernel(bases, counts):
    info = plsc.get_sparse_core_info()
    nc, ns = info.num_cores, info.num_subcores
    nw = nc * ns

    mesh = plsc.VectorSubcoreMesh(core_axis_name="c", subcore_axis_name="s")

    @functools.partial(
        pl.kernel,
        mesh=mesh,
        out_type=jax.ShapeDtypeStruct((_B * _NTOK, _D), jnp.float32),
        scratch_types=[
            pltpu.VMEM((2, _CHUNK), jnp.int32),
            pltpu.VMEM((_CHUNK, _D), jnp.float32),
            pltpu.SemaphoreType.DMA,
        ],
    )
    def k(x0, x1, x2, x3, x4, tbl, idx_hbm, out, idxv, datav, sem):
        wid = lax.axis_index("s") * nc + lax.axis_index("c")
        srcs = (x0, x1, x2, x3, x4, tbl)
        for g in range(6):
            n, base, srcref = counts[g], bases[g], srcs[g]
            trips = jnp.maximum(0, (n - wid + nw - 1) // nw)

            def body(i, carry, base=base, srcref=srcref):
                c = base + wid + i * nw
                pltpu.sync_copy(idx_hbm.at[c], idxv)
                pltpu.async_copy(srcref.at[idxv.at[0]], datav, sem).wait()
                pltpu.async_copy(datav, out.at[idxv.at[1]], sem).wait()
                return carry

            lax.fori_loop(0, trips, body, 0)

    return k


def kernel(ch1v, ch2v, dcv, ch3v, ch3c, sp_token_table, rnd_token_table):
    idx, bases, counts, seg_index = _plan()
    segs = [x.reshape(_B * _T, _D) for x in (ch1v, ch2v, dcv, ch3v, ch3c)]
    tbl = jnp.concatenate([sp_token_table, rnd_token_table], axis=0)
    k = _make_sc_kernel(bases, counts)
    out2d = k(*segs, tbl, jnp.asarray(idx))
    return out2d.reshape(_B, _NTOK, _D), jnp.asarray(seg_index)


# speedup vs baseline: 1.4055x; 1.4055x over previous
import functools

import numpy as np
import jax
import jax.numpy as jnp
from jax import lax
from jax.experimental import pallas as pl
from jax.experimental.pallas import tpu as pltpu
from jax.experimental.pallas import tpu_sc as plsc

_B, _T, _D, _SEG, _R = 64, 512, 256, 5, 100
_MASK_RATE = 0.15
_NTOK = _SEG * _T + 7
_OFFS = (2, 515, 1028, 1541, 2054)
_SPECIALS = ((0, 4), (1, 1), (514, 3), (1027, 3), (1540, 3), (2053, 3), (2566, 2))
_CHUNK = 128

_PLAN_CACHE = None

_ROT1 = (13, 15, 26, 6)
_ROT2 = (17, 29, 16, 24)


def _rotl(x, d):
    return ((x << np.uint32(d)) | (x >> np.uint32(32 - d))).astype(np.uint32)


def _tf_pair(key, c1, c2):
    x0 = c1.astype(np.uint32).copy()
    x1 = c2.astype(np.uint32).copy()
    ks0, ks1 = np.uint32(key[0]), np.uint32(key[1])
    ks2 = np.uint32(ks0 ^ ks1 ^ np.uint32(0x1BD11BDA))
    x0 += ks0
    x1 += ks1

    def rounds(x0, x1, rots):
        for r in rots:
            x0 = (x0 + x1).astype(np.uint32)
            x1 = _rotl(x1, r)
            x1 = x1 ^ x0
        return x0, x1

    x0, x1 = rounds(x0, x1, _ROT1)
    x0 = (x0 + ks1).astype(np.uint32); x1 = (x1 + ks2 + np.uint32(1)).astype(np.uint32)
    x0, x1 = rounds(x0, x1, _ROT2)
    x0 = (x0 + ks2).astype(np.uint32); x1 = (x1 + ks0 + np.uint32(2)).astype(np.uint32)
    x0, x1 = rounds(x0, x1, _ROT1)
    x0 = (x0 + ks0).astype(np.uint32); x1 = (x1 + ks1 + np.uint32(3)).astype(np.uint32)
    x0, x1 = rounds(x0, x1, _ROT2)
    x0 = (x0 + ks1).astype(np.uint32); x1 = (x1 + ks2 + np.uint32(4)).astype(np.uint32)
    x0, x1 = rounds(x0, x1, _ROT1)
    x0 = (x0 + ks2).astype(np.uint32); x1 = (x1 + ks0 + np.uint32(5)).astype(np.uint32)
    return x0, x1


def _np_split(key, num):
    i = np.arange(num, dtype=np.uint64)
    b1, b2 = _tf_pair(key, (i >> np.uint64(32)).astype(np.uint32), i.astype(np.uint32))
    return np.stack([b1, b2], axis=1)


def _np_random_bits(key, shape):
    n = int(np.prod(shape))
    i = np.arange(n, dtype=np.uint64)
    b1, b2 = _tf_pair(key, (i >> np.uint64(32)).astype(np.uint32), i.astype(np.uint32))
    return (b1 ^ b2).reshape(shape)


def _np_uniform(key, shape):
    bits = _np_random_bits(key, shape)
    f = ((bits >> np.uint32(9)) | np.uint32(0x3F800000)).view(np.float32) - np.float32(1.0)
    return np.maximum(np.float32(0.0), f)


def _np_randint(key, shape, minval, maxval):
    k1, k2 = _np_split(key, 2)
    higher = _np_random_bits(k1, shape)
    lower = _np_random_bits(k2, shape)
    span = np.uint32(maxval - minval)
    multiplier = np.uint32((2 ** 16) % int(span))
    multiplier = np.uint32((int(multiplier) * int(multiplier)) % int(span))
    offset = ((higher % span) * multiplier + lower % span) % span
    return (minval + offset).astype(np.int32)


def _plan():
    global _PLAN_CACHE
    if _PLAN_CACHE is not None:
        return _PLAN_CACHE
    ks = _np_split(np.array([0, 42], np.uint32), 4)
    u1 = _np_uniform(ks[0], (_SEG, _T))
    u2 = _np_uniform(ks[1], (_SEG, _T))
    u3 = _np_uniform(ks[2], (_SEG, _T))
    ridx = _np_randint(ks[3], (_SEG, _T), 0, _R)
    replace = (u1 < _MASK_RATE) & (np.arange(_T)[None, :] < _T - 1)
    choice = np.where(u2 < 0.8, 0, np.where(u3 < 0.5, 1, 2))
    overwrite = replace & (choice != 1)
    tblrow = np.where(choice == 0, 5, 6 + ridx)

    js = [j for j, _ in _SPECIALS]
    ks = [kk for _, kk in _SPECIALS]
    for s in range(_SEG):
        ot = np.nonzero(overwrite[s])[0]
        js += list(_OFFS[s] + ot)
        ks += list(tblrow[s][ot])
    js = np.array(js)
    ks = np.array(ks)
    m = len(js)
    pad = (-m) % _CHUNK
    js = np.concatenate([js, np.full(pad, js[-1])])
    ks = np.concatenate([ks, np.full(pad, ks[-1])])
    nch = len(js) // _CHUNK

    bidx = np.arange(_B)
    src = np.broadcast_to(ks[None, :], (_B, len(ks)))
    dst = bidx[:, None] * _NTOK + js[None, :]
    idx_tbl = np.stack([src, dst], axis=1).reshape(_B, 2, nch, _CHUNK)
    idx_tbl = np.ascontiguousarray(idx_tbl.transpose(0, 2, 1, 3)).astype(np.int32)

    t = np.arange(_T)
    dstb = (bidx[:, None, None] * _NTOK
            + np.asarray(_OFFS)[None, :, None] + t[None, None, :])
    idx_bulk = np.ascontiguousarray(
        dstb.reshape(_B, _SEG * (_T // _CHUNK), _CHUNK)).astype(np.int32)

    seg_index = np.array([0] * 515 + sum(([s] * 513 for s in range(1, 5)), []),
                         dtype=np.int32)
    _PLAN_CACHE = (idx_tbl, idx_bulk, nch, seg_index)
    return _PLAN_CACHE


@functools.lru_cache(maxsize=1)
def _make_sc_kernel(nch):
    info = plsc.get_sparse_core_info()
    nc, ns = info.num_cores, info.num_subcores
    nw = nc * ns
    bpw = _B // nw

    mesh = plsc.VectorSubcoreMesh(core_axis_name="c", subcore_axis_name="s")

    nbuf = 3
    cps = _T // _CHUNK
    nbulk = bpw * _SEG * cps

    @functools.partial(
        pl.kernel,
        mesh=mesh,
        out_type=jax.ShapeDtypeStruct((_B * _NTOK, _D), jnp.float32),
        scratch_types=[
            pltpu.VMEM((bpw, _SEG * cps, _CHUNK), jnp.int32),
            pltpu.VMEM((bpw, nch, 2, _CHUNK), jnp.int32),
            pltpu.VMEM((nbuf, _CHUNK, _D), jnp.float32),
            pltpu.SemaphoreType.DMA,
            pltpu.SemaphoreType.DMA,
            pltpu.SemaphoreType.DMA,
            pltpu.SemaphoreType.DMA,
        ],
    )
    def k(x0, x1, x2, x3, x4, tbl, idxb_hbm, idxt_hbm, out,
          bslab, tslab, bufs, lsem, s0, s1, s2):
        wid = lax.axis_index("s") * nc + lax.axis_index("c")
        b0 = wid * bpw
        segs = (x0, x1, x2, x3, x4)
        ssems = (s0, s1, s2)
        for i in range(bpw):
            pltpu.sync_copy(idxb_hbm.at[b0 + i], bslab.at[i])
            pltpu.sync_copy(idxt_hbm.at[b0 + i], tslab.at[i])

        def load(ci, slot):
            i, r = divmod(ci, _SEG * cps)
            s, c = divmod(r, cps)
            b = b0 + i
            return pltpu.async_copy(
                segs[s].at[pl.ds(b * _T + c * _CHUNK, _CHUNK)],
                bufs.at[slot], lsem)

        def scatter(ci, slot):
            i, r = divmod(ci, _SEG * cps)
            return pltpu.async_copy(
                bufs.at[slot], out.at[bslab.at[i, r]], ssems[slot])

        lh = [None] * nbulk
        sh = [None] * nbulk
        for ph in range(nbulk + 1):
            if ph < nbulk:
                slot = ph % nbuf
                if ph >= nbuf:
                    sh[ph - nbuf].wait()
                lh[ph] = load(ph, slot)
            if ph >= 1:
                j = ph - 1
                lh[j].wait()
                sh[j] = scatter(j, j % nbuf)
        for j in range(nbulk - nbuf, nbulk):
            sh[j].wait()

        ghs = [
            pltpu.async_copy(tbl.at[tslab.at[0, j, 0]], bufs.at[j], lsem)
            for j in range(nch)
        ]
        for h in ghs:
            h.wait()
        shs = []
        for i in range(bpw):
            for j in range(nch):
                shs.append(pltpu.async_copy(
                    bufs.at[j], out.at[tslab.at[i, j, 1]], ssems[j]))
        for h in shs:
            h.wait()

    return k


def kernel(ch1v, ch2v, dcv, ch3v, ch3c, sp_token_table, rnd_token_table):
    idx_tbl, idx_bulk, nch, seg_index = _plan()
    segs = [x.reshape(_B * _T, _D) for x in (ch1v, ch2v, dcv, ch3v, ch3c)]
    tbl = jnp.concatenate([sp_token_table, rnd_token_table], axis=0)
    k = _make_sc_kernel(nch)
    out2d = k(*segs, tbl, jnp.asarray(idx_bulk), jnp.asarray(idx_tbl))
    return out2d.reshape(_B, _NTOK, _D), jnp.asarray(seg_index)


# speedup vs baseline: 1.4131x; 1.0054x over previous
import functools

import numpy as np
import jax
import jax.numpy as jnp
from jax import lax
from jax.experimental import pallas as pl
from jax.experimental.pallas import tpu as pltpu
from jax.experimental.pallas import tpu_sc as plsc

_B, _T, _D, _SEG, _R = 64, 512, 256, 5, 100
_MASK_RATE = 0.15
_NTOK = _SEG * _T + 7
_OFFS = (2, 515, 1028, 1541, 2054)
_SPECIALS = ((0, 4), (1, 1), (514, 3), (1027, 3), (1540, 3), (2053, 3), (2566, 2))
_CHUNK = 128

_PLAN_CACHE = None

_ROT1 = (13, 15, 26, 6)
_ROT2 = (17, 29, 16, 24)


def _rotl(x, d):
    return ((x << np.uint32(d)) | (x >> np.uint32(32 - d))).astype(np.uint32)


def _tf_pair(key, c1, c2):
    x0 = c1.astype(np.uint32).copy()
    x1 = c2.astype(np.uint32).copy()
    ks0, ks1 = np.uint32(key[0]), np.uint32(key[1])
    ks2 = np.uint32(ks0 ^ ks1 ^ np.uint32(0x1BD11BDA))
    x0 += ks0
    x1 += ks1

    def rounds(x0, x1, rots):
        for r in rots:
            x0 = (x0 + x1).astype(np.uint32)
            x1 = _rotl(x1, r)
            x1 = x1 ^ x0
        return x0, x1

    x0, x1 = rounds(x0, x1, _ROT1)
    x0 = (x0 + ks1).astype(np.uint32); x1 = (x1 + ks2 + np.uint32(1)).astype(np.uint32)
    x0, x1 = rounds(x0, x1, _ROT2)
    x0 = (x0 + ks2).astype(np.uint32); x1 = (x1 + ks0 + np.uint32(2)).astype(np.uint32)
    x0, x1 = rounds(x0, x1, _ROT1)
    x0 = (x0 + ks0).astype(np.uint32); x1 = (x1 + ks1 + np.uint32(3)).astype(np.uint32)
    x0, x1 = rounds(x0, x1, _ROT2)
    x0 = (x0 + ks1).astype(np.uint32); x1 = (x1 + ks2 + np.uint32(4)).astype(np.uint32)
    x0, x1 = rounds(x0, x1, _ROT1)
    x0 = (x0 + ks2).astype(np.uint32); x1 = (x1 + ks0 + np.uint32(5)).astype(np.uint32)
    return x0, x1


def _np_split(key, num):
    i = np.arange(num, dtype=np.uint64)
    b1, b2 = _tf_pair(key, (i >> np.uint64(32)).astype(np.uint32), i.astype(np.uint32))
    return np.stack([b1, b2], axis=1)


def _np_random_bits(key, shape):
    n = int(np.prod(shape))
    i = np.arange(n, dtype=np.uint64)
    b1, b2 = _tf_pair(key, (i >> np.uint64(32)).astype(np.uint32), i.astype(np.uint32))
    return (b1 ^ b2).reshape(shape)


def _np_uniform(key, shape):
    bits = _np_random_bits(key, shape)
    f = ((bits >> np.uint32(9)) | np.uint32(0x3F800000)).view(np.float32) - np.float32(1.0)
    return np.maximum(np.float32(0.0), f)


def _np_randint(key, shape, minval, maxval):
    k1, k2 = _np_split(key, 2)
    higher = _np_random_bits(k1, shape)
    lower = _np_random_bits(k2, shape)
    span = np.uint32(maxval - minval)
    multiplier = np.uint32((2 ** 16) % int(span))
    multiplier = np.uint32((int(multiplier) * int(multiplier)) % int(span))
    offset = ((higher % span) * multiplier + lower % span) % span
    return (minval + offset).astype(np.int32)


def _plan():
    global _PLAN_CACHE
    if _PLAN_CACHE is not None:
        return _PLAN_CACHE
    ks = _np_split(np.array([0, 42], np.uint32), 4)
    u1 = _np_uniform(ks[0], (_SEG, _T))
    u2 = _np_uniform(ks[1], (_SEG, _T))
    u3 = _np_uniform(ks[2], (_SEG, _T))
    ridx = _np_randint(ks[3], (_SEG, _T), 0, _R)
    replace = (u1 < _MASK_RATE) & (np.arange(_T)[None, :] < _T - 1)
    choice = np.where(u2 < 0.8, 0, np.where(u3 < 0.5, 1, 2))
    overwrite = replace & (choice != 1)
    tblrow = np.where(choice == 0, 5, 6 + ridx)

    js = [j for j, _ in _SPECIALS]
    ks = [kk for _, kk in _SPECIALS]
    for s in range(_SEG):
        ot = np.nonzero(overwrite[s])[0]
        js += list(_OFFS[s] + ot)
        ks += list(tblrow[s][ot])
    js = np.array(js)
    ks = np.array(ks)
    m = len(js)
    pad = (-m) % _CHUNK
    js = np.concatenate([js, np.full(pad, js[-1])])
    ks = np.concatenate([ks, np.full(pad, ks[-1])])
    nch = len(js) // _CHUNK

    tpat = np.ascontiguousarray(
        np.stack([ks, js], 0).reshape(2, nch, _CHUNK).transpose(1, 0, 2)
    ).astype(np.int32)

    seg_index = np.array([0] * 515 + sum(([s] * 513 for s in range(1, 5)), []),
                         dtype=np.int32)
    _PLAN_CACHE = (tpat, nch, seg_index)
    return _PLAN_CACHE


@functools.lru_cache(maxsize=1)
def _make_sc_kernel(nch):
    info = plsc.get_sparse_core_info()
    nc, ns = info.num_cores, info.num_subcores
    nw = nc * ns
    bpw = _B // nw

    mesh = plsc.VectorSubcoreMesh(core_axis_name="c", subcore_axis_name="s")

    nbuf = 3
    cps = _T // _CHUNK
    nbulk = bpw * _SEG * cps

    @functools.partial(
        pl.kernel,
        mesh=mesh,
        out_type=jax.ShapeDtypeStruct((_B * _NTOK, _D), jnp.float32),
        scratch_types=[
            pltpu.VMEM((bpw, _SEG * cps, _CHUNK), jnp.int32),
            pltpu.VMEM((nch, 2, _CHUNK), jnp.int32),
            pltpu.VMEM((bpw, nch, _CHUNK), jnp.int32),
            pltpu.VMEM((nbuf, _CHUNK, _D), jnp.float32),
            pltpu.SemaphoreType.DMA,
            pltpu.SemaphoreType.DMA,
            pltpu.SemaphoreType.DMA,
            pltpu.SemaphoreType.DMA,
        ],
    )
    def k(x0, x1, x2, x3, x4, tbl, tpat_hbm, out,
          bslab, tpat, tslab, bufs, lsem, s0, s1, s2):
        wid = lax.axis_index("s") * nc + lax.axis_index("c")
        b0 = wid * bpw
        segs = (x0, x1, x2, x3, x4)
        ssems = (s0, s1, s2)
        pltpu.sync_copy(tpat_hbm, tpat)
        iota16 = lax.iota(jnp.int32, 16)
        for i in range(bpw):
            b = b0 + i
            for s in range(_SEG):
                for c in range(cps):
                    base = b * _NTOK + _OFFS[s] + c * _CHUNK
                    for q in range(_CHUNK // 16):
                        bslab[i, s * cps + c, pl.ds(q * 16, 16)] = (
                            iota16 + (base + q * 16))
            for j in range(nch):
                for q in range(_CHUNK // 16):
                    tslab[i, j, pl.ds(q * 16, 16)] = (
                        tpat[j, 1, pl.ds(q * 16, 16)] + b * _NTOK)

        def load(ci, slot):
            i, r = divmod(ci, _SEG * cps)
            s, c = divmod(r, cps)
            b = b0 + i
            return pltpu.async_copy(
                segs[s].at[pl.ds(b * _T + c * _CHUNK, _CHUNK)],
                bufs.at[slot], lsem)

        def scatter(ci, slot):
            i, r = divmod(ci, _SEG * cps)
            return pltpu.async_copy(
                bufs.at[slot], out.at[bslab.at[i, r]], ssems[slot])

        la = 2
        lh = [None] * nbulk
        sh = [None] * nbulk
        for ph in range(nbulk + la):
            if ph < nbulk:
                slot = ph % nbuf
                if ph >= nbuf:
                    sh[ph - nbuf].wait()
                lh[ph] = load(ph, slot)
            if ph >= la:
                j = ph - la
                lh[j].wait()
                sh[j] = scatter(j, j % nbuf)
        for j in range(nbulk - nbuf, nbulk):
            sh[j].wait()

        ghs = [
            pltpu.async_copy(tbl.at[tpat.at[j, 0]], bufs.at[j], lsem)
            for j in range(nch)
        ]
        for h in ghs:
            h.wait()
        shs = []
        for i in range(bpw):
            for j in range(nch):
                shs.append(pltpu.async_copy(
                    bufs.at[j], out.at[tslab.at[i, j]], ssems[j]))
        for h in shs:
            h.wait()

    return k


def kernel(ch1v, ch2v, dcv, ch3v, ch3c, sp_token_table, rnd_token_table):
    tpat, nch, seg_index = _plan()
    segs = [x.reshape(_B * _T, _D) for x in (ch1v, ch2v, dcv, ch3v, ch3c)]
    tbl = jnp.concatenate([sp_token_table, rnd_token_table], axis=0)
    k = _make_sc_kernel(nch)
    out2d = k(*segs, tbl, jnp.asarray(tpat))
    return out2d.reshape(_B, _NTOK, _D), jnp.asarray(seg_index)


# speedup vs baseline: 2.0168x; 1.4272x over previous
import functools

import numpy as np
import jax
import jax.numpy as jnp
from jax import lax
from jax.experimental import pallas as pl
from jax.experimental.pallas import tpu as pltpu
from jax.experimental.pallas import tpu_sc as plsc

_B, _T, _D, _SEG, _R = 64, 512, 256, 5, 100
_MASK_RATE = 0.15
_NTOK = _SEG * _T + 7
_OFFS = (2, 515, 1028, 1541, 2054)
_SPECIALS = ((0, 4), (1, 1), (514, 3), (1027, 3), (1540, 3), (2053, 3), (2566, 2))
_CHUNK = 128

_PLAN_CACHE = None

_ROT1 = (13, 15, 26, 6)
_ROT2 = (17, 29, 16, 24)


def _rotl(x, d):
    return ((x << np.uint32(d)) | (x >> np.uint32(32 - d))).astype(np.uint32)


def _tf_pair(key, c1, c2):
    x0 = c1.astype(np.uint32).copy()
    x1 = c2.astype(np.uint32).copy()
    ks0, ks1 = np.uint32(key[0]), np.uint32(key[1])
    ks2 = np.uint32(ks0 ^ ks1 ^ np.uint32(0x1BD11BDA))
    x0 += ks0
    x1 += ks1

    def rounds(x0, x1, rots):
        for r in rots:
            x0 = (x0 + x1).astype(np.uint32)
            x1 = _rotl(x1, r)
            x1 = x1 ^ x0
        return x0, x1

    x0, x1 = rounds(x0, x1, _ROT1)
    x0 = (x0 + ks1).astype(np.uint32); x1 = (x1 + ks2 + np.uint32(1)).astype(np.uint32)
    x0, x1 = rounds(x0, x1, _ROT2)
    x0 = (x0 + ks2).astype(np.uint32); x1 = (x1 + ks0 + np.uint32(2)).astype(np.uint32)
    x0, x1 = rounds(x0, x1, _ROT1)
    x0 = (x0 + ks0).astype(np.uint32); x1 = (x1 + ks1 + np.uint32(3)).astype(np.uint32)
    x0, x1 = rounds(x0, x1, _ROT2)
    x0 = (x0 + ks1).astype(np.uint32); x1 = (x1 + ks2 + np.uint32(4)).astype(np.uint32)
    x0, x1 = rounds(x0, x1, _ROT1)
    x0 = (x0 + ks2).astype(np.uint32); x1 = (x1 + ks0 + np.uint32(5)).astype(np.uint32)
    return x0, x1


def _np_split(key, num):
    i = np.arange(num, dtype=np.uint64)
    b1, b2 = _tf_pair(key, (i >> np.uint64(32)).astype(np.uint32), i.astype(np.uint32))
    return np.stack([b1, b2], axis=1)


def _np_random_bits(key, shape):
    n = int(np.prod(shape))
    i = np.arange(n, dtype=np.uint64)
    b1, b2 = _tf_pair(key, (i >> np.uint64(32)).astype(np.uint32), i.astype(np.uint32))
    return (b1 ^ b2).reshape(shape)


def _np_uniform(key, shape):
    bits = _np_random_bits(key, shape)
    f = ((bits >> np.uint32(9)) | np.uint32(0x3F800000)).view(np.float32) - np.float32(1.0)
    return np.maximum(np.float32(0.0), f)


def _np_randint(key, shape, minval, maxval):
    k1, k2 = _np_split(key, 2)
    higher = _np_random_bits(k1, shape)
    lower = _np_random_bits(k2, shape)
    span = np.uint32(maxval - minval)
    multiplier = np.uint32((2 ** 16) % int(span))
    multiplier = np.uint32((int(multiplier) * int(multiplier)) % int(span))
    offset = ((higher % span) * multiplier + lower % span) % span
    return (minval + offset).astype(np.int32)


def _plan():
    global _PLAN_CACHE
    if _PLAN_CACHE is not None:
        return _PLAN_CACHE
    ks = _np_split(np.array([0, 42], np.uint32), 4)
    u1 = _np_uniform(ks[0], (_SEG, _T))
    u2 = _np_uniform(ks[1], (_SEG, _T))
    u3 = _np_uniform(ks[2], (_SEG, _T))
    ridx = _np_randint(ks[3], (_SEG, _T), 0, _R)
    replace = (u1 < _MASK_RATE) & (np.arange(_T)[None, :] < _T - 1)
    choice = np.where(u2 < 0.8, 0, np.where(u3 < 0.5, 1, 2))
    overwrite = replace & (choice != 1)
    tblrow = np.where(choice == 0, 5, 6 + ridx)

    js = [j for j, _ in _SPECIALS]
    kk = [k for _, k in _SPECIALS]
    for s in range(_SEG):
        ot = np.nonzero(overwrite[s])[0]
        js += list(_OFFS[s] + ot)
        kk += list(tblrow[s][ot])
    js = np.array(js)
    kk = np.array(kk)
    pad = (-len(js)) % _CHUNK
    js = np.concatenate([js, np.full(pad, js[-1])])
    kk = np.concatenate([kk, np.full(pad, kk[-1])])
    nch = len(js) // _CHUNK

    tpat = np.ascontiguousarray(
        np.stack([kk, js * _B], 0).reshape(2, nch, _CHUNK).transpose(1, 0, 2)
    ).astype(np.int32)

    seg_index = np.array([0] * 515 + sum(([s] * 513 for s in range(1, 5)), []),
                         dtype=np.int32)
    _PLAN_CACHE = (tpat, nch, seg_index)
    return _PLAN_CACHE


@functools.lru_cache(maxsize=1)
def _make_sc_kernel(nch):
    info = plsc.get_sparse_core_info()
    nc, ns = info.num_cores, info.num_subcores
    nw = nc * ns
    bpw = _B // nw

    mesh = plsc.VectorSubcoreMesh(core_axis_name="c", subcore_axis_name="s")

    nbuf = 3
    cps = _T // _CHUNK
    nbulk = bpw * _SEG * cps

    @functools.partial(
        pl.kernel,
        mesh=mesh,
        out_type=jax.ShapeDtypeStruct((_B * _NTOK, _D), jnp.float32),
        scratch_types=[
            pltpu.VMEM((bpw, _SEG * cps, _CHUNK), jnp.int32),
            pltpu.VMEM((nch, 2, _CHUNK), jnp.int32),
            pltpu.VMEM((bpw, nch, _CHUNK), jnp.int32),
            pltpu.VMEM((nbuf, _CHUNK, _D), jnp.float32),
            pltpu.SemaphoreType.DMA,
            pltpu.SemaphoreType.DMA,
            pltpu.SemaphoreType.DMA,
            pltpu.SemaphoreType.DMA,
        ],
    )
    def k(x0, x1, x2, x3, x4, tbl, tpat_hbm, out,
          bslab, tpat, tslab, bufs, lsem, s0, s1, s2):
        wid = lax.axis_index("s") * nc + lax.axis_index("c")
        b0 = wid * bpw
        segs = (x0, x1, x2, x3, x4)
        ssems = (s0, s1, s2)
        pltpu.sync_copy(tpat_hbm, tpat)
        iota16b = lax.iota(jnp.int32, 16) * _B
        for i in range(bpw):
            b = b0 + i
            for s in range(_SEG):
                for c in range(cps):
                    base = (_OFFS[s] + c * _CHUNK) * _B + b
                    for q in range(_CHUNK // 16):
                        bslab[i, s * cps + c, pl.ds(q * 16, 16)] = (
                            iota16b + (base + q * 16 * _B))
            for j in range(nch):
                for q in range(_CHUNK // 16):
                    tslab[i, j, pl.ds(q * 16, 16)] = (
                        tpat[j, 1, pl.ds(q * 16, 16)] + b)

        def load(ci, slot):
            i, r = divmod(ci, _SEG * cps)
            s, c = divmod(r, cps)
            b = b0 + i
            return pltpu.async_copy(
                segs[s].at[pl.ds(b * _T + c * _CHUNK, _CHUNK)],
                bufs.at[slot], lsem)

        def scatter(ci, slot):
            i, r = divmod(ci, _SEG * cps)
            return pltpu.async_copy(
                bufs.at[slot], out.at[bslab.at[i, r]], ssems[slot])

        la = 2
        lh = [None] * nbulk
        sh = [None] * nbulk
        for ph in range(nbulk + la):
            if ph < nbulk:
                slot = ph % nbuf
                if ph >= nbuf:
                    sh[ph - nbuf].wait()
                lh[ph] = load(ph, slot)
            if ph >= la:
                j = ph - la
                lh[j].wait()
                sh[j] = scatter(j, j % nbuf)
        for j in range(nbulk - nbuf, nbulk):
            sh[j].wait()

        ghs = [
            pltpu.async_copy(tbl.at[tpat.at[j, 0]], bufs.at[j], lsem)
            for j in range(nch)
        ]
        for h in ghs:
            h.wait()
        shs = []
        for i in range(bpw):
            for j in range(nch):
                shs.append(pltpu.async_copy(
                    bufs.at[j], out.at[tslab.at[i, j]], ssems[j]))
        for h in shs:
            h.wait()

    return k


def kernel(ch1v, ch2v, dcv, ch3v, ch3c, sp_token_table, rnd_token_table):
    tpat, nch, seg_index = _plan()
    segs = [x.reshape(_B * _T, _D) for x in (ch1v, ch2v, dcv, ch3v, ch3c)]
    tbl = jnp.concatenate([sp_token_table, rnd_token_table], axis=0)
    k = _make_sc_kernel(nch)
    out2d = k(*segs, tbl, jnp.asarray(tpat))
    out = jnp.transpose(out2d.reshape(_NTOK, _B, _D), (1, 0, 2))
    return out, jnp.asarray(seg_index)
